# baseline (speedup 1.0000x reference)
import functools

import jax
import jax.numpy as jnp
from jax import lax
from jax.experimental import pallas as pl
from jax.experimental.pallas import tpu as pltpu
from jax.experimental.pallas import tpu_sc as plsc

VOCAB = 50
D = 16
B = 16384
L = 200
N = B * L
RCOLS = 512
R = N // RCOLS
NC = 2
NS = 16
NW = NC * NS
RW = R // NW
CHUNK = 4
NCHUNK = RW // CHUNK


def _body(tags_ref, table_ref, out_ref, relu_ref, tab_v, idx_v, rows_v,
          si0, si1, sg0, sg1, so0, so1):
    wid = lax.axis_index("s") * NC + lax.axis_index("c")
    si = (si0, si1)
    sg = (sg0, sg1)
    so = (so0, so1)

    pltpu.sync_copy(table_ref, tab_v)
    for i in range(VOCAB):
        tab_v[i, :] = jnp.maximum(tab_v[i, :], 0.0)
    pltpu.sync_copy(tab_v, relu_ref)

    def rbase(c):
        return wid * RW + c * CHUNK

    def issue_idx(c, b):
        pltpu.async_copy(tags_ref.at[pl.ds(rbase(c), CHUNK)], idx_v.at[b], si[b])

    def wait_idx(b):
        pltpu.make_async_copy(tags_ref.at[pl.ds(0, CHUNK)], idx_v.at[b], si[b]).wait()

    def fire_g(b):
        for k in range(CHUNK):
            pltpu.async_copy(relu_ref.at[idx_v.at[b, k]], rows_v.at[b, k], sg[b])

    def wait_g(b):
        for k in range(CHUNK):
            pltpu.make_async_copy(out_ref.at[0], rows_v.at[b, k], sg[b]).wait()

    def issue_out(c, b):
        pltpu.async_copy(rows_v.at[b], out_ref.at[pl.ds(rbase(c), CHUNK)], so[b])

    def wait_out(b):
        pltpu.make_async_copy(rows_v.at[b], out_ref.at[pl.ds(0, CHUNK)], so[b]).wait()

    issue_idx(0, 0)
    issue_idx(1, 1)
    wait_idx(0)
    fire_g(0)

    @pl.loop(0, NCHUNK // 2)
    def _super(s):
        for j in range(2):
            c = s * 2 + j
            b = j
            nb = 1 - j

            @pl.when(jnp.logical_and(c >= 1, c + 1 < NCHUNK))
            def _():
                wait_out(nb)

            @pl.when(c + 1 < NCHUNK)
            def _():
                wait_idx(nb)
                fire_g(nb)

            wait_g(b)
            issue_out(c, b)

            @pl.when(c + 2 < NCHUNK)
            def _():
                issue_idx(c + 2, b)

    wait_out(0)
    wait_out(1)


@jax.jit
def _run(tags2d, embedding):
    mesh = plsc.VectorSubcoreMesh(
        core_axis_name="c", subcore_axis_name="s", num_cores=NC, num_subcores=NS
    )
    kern = pl.kernel(
        _body,
        out_type=(
            jax.ShapeDtypeStruct((R, RCOLS, D), jnp.float32),
            jax.ShapeDtypeStruct((VOCAB, D), jnp.float32),
        ),
        mesh=mesh,
        scratch_types=[
            pltpu.VMEM((VOCAB, D), jnp.float32),
            pltpu.VMEM((2, CHUNK, RCOLS), jnp.int32),
            pltpu.VMEM((2, CHUNK, RCOLS, D), jnp.float32),
            pltpu.SemaphoreType.DMA,
            pltpu.SemaphoreType.DMA,
            pltpu.SemaphoreType.DMA,
            pltpu.SemaphoreType.DMA,
            pltpu.SemaphoreType.DMA,
            pltpu.SemaphoreType.DMA,
        ],
        compiler_params=pltpu.CompilerParams(use_tc_tiling_on_sc=False),
    )
    out3, _ = kern(tags2d, embedding)
    return out3


def kernel(preprocessed_tags, embedding):
    tags2d = preprocessed_tags.reshape(R, RCOLS)
    out3 = _run(tags2d, embedding)
    return out3.reshape(B, L, D)


# speedup vs baseline: 1.6218x; 1.6218x over previous
import functools

import jax
import jax.numpy as jnp
from jax import lax
from jax.experimental import pallas as pl
from jax.experimental.pallas import tpu as pltpu
from jax.experimental.pallas import tpu_sc as plsc

VOCAB = 50
D = 16
B = 16384
L = 200
N = B * L
NC = 2
NS = 16
NW = NC * NS
IDXW = N // NW
CHUNK = 2048
NCHUNK = IDXW // CHUNK
GROUPS = CHUNK // 16


def _body(tags_ref, table_ref, out_ref, tab_v, idx_v, out_v, si0, si1, so0, so1):
    wid = lax.axis_index("s") * NC + lax.axis_index("c")
    si = (si0, si1)
    so = (so0, so1)

    pltpu.sync_copy(table_ref, tab_v)
    for i in range(VOCAB):
        tab_v[i, :] = jnp.maximum(tab_v[i, :], 0.0)

    lane = lax.iota(jnp.int32, 16)
    row_off = lane * D

    def ibase(c):
        return wid * IDXW + c * CHUNK

    def issue_idx(c, b):
        pltpu.async_copy(tags_ref.at[pl.ds(ibase(c), CHUNK)], idx_v.at[b], si[b])

    def wait_idx(b):
        pltpu.make_async_copy(tags_ref.at[pl.ds(0, CHUNK)], idx_v.at[b], si[b]).wait()

    def issue_out(c, b):
        pltpu.async_copy(out_v.at[b], out_ref.at[pl.ds(ibase(c) * D, CHUNK * D)], so[b])

    def wait_out(b):
        pltpu.make_async_copy(out_v.at[b], out_ref.at[pl.ds(0, CHUNK * D)], so[b]).wait()

    issue_idx(0, 0)
    issue_idx(1, 1)

    @pl.loop(0, NCHUNK // 2)
    def _super(s):
        for b in range(2):
            c = s * 2 + b

            wait_idx(b)

            @pl.when(c >= 2)
            def _():
                wait_out(b)

            @pl.loop(0, GROUPS)
            def _grp(g):
                iv = idx_v[b, pl.ds(g * 16, 16)]
                dst = g * (16 * D) + row_off
                for d in range(D):
                    vals = plsc.load_gather(
                        tab_v, [iv, jnp.full((16,), d, jnp.int32)]
                    )
                    plsc.store_scatter(out_v.at[b], [dst + d], vals)

            issue_out(c, b)

            @pl.when(c + 2 < NCHUNK)
            def _():
                issue_idx(c + 2, b)

    wait_out(0)
    wait_out(1)


@jax.jit
def _run(tags1d, embedding):
    mesh = plsc.VectorSubcoreMesh(
        core_axis_name="c", subcore_axis_name="s", num_cores=NC, num_subcores=NS
    )
    kern = pl.kernel(
        _body,
        out_type=jax.ShapeDtypeStruct((N * D,), jnp.float32),
        mesh=mesh,
        scratch_types=[
            pltpu.VMEM((VOCAB, D), jnp.float32),
            pltpu.VMEM((2, CHUNK), jnp.int32),
            pltpu.VMEM((2, CHUNK * D), jnp.float32),
            pltpu.SemaphoreType.DMA,
            pltpu.SemaphoreType.DMA,
            pltpu.SemaphoreType.DMA,
            pltpu.SemaphoreType.DMA,
        ],
        compiler_params=pltpu.CompilerParams(
            use_tc_tiling_on_sc=False, needs_layout_passes=False
        ),
    )
    return kern(tags1d, embedding)


def kernel(preprocessed_tags, embedding):
    tags1d = preprocessed_tags.reshape(N)
    out = _run(tags1d, embedding)
    return out.reshape(B, L, D)


# speedup vs baseline: 1.6242x; 1.0015x over previous
import functools

import jax
import jax.numpy as jnp
from jax import lax
from jax.experimental import pallas as pl
from jax.experimental.pallas import tpu as pltpu
from jax.experimental.pallas import tpu_sc as plsc

VOCAB = 50
D = 16
B = 16384
L = 200
N = B * L
NC = 2
NS = 16
NW = NC * NS
IDXW = N // NW
CHUNK = 2048
NCHUNK = IDXW // CHUNK
GROUPS = CHUNK // 16


def _body(tags_ref, table_ref, out_ref, tab_v, idx_v, out_v, si0, si1, so0, so1):
    wid = lax.axis_index("s") * NC + lax.axis_index("c")
    si = (si0, si1)
    so = (so0, so1)

    pltpu.sync_copy(table_ref, tab_v)
    for i in range(VOCAB):
        tab_v[i, :] = jnp.maximum(tab_v[i, :], 0.0)

    lane = lax.iota(jnp.int32, 16)
    row_off = lane * D

    def ibase(c):
        return wid * IDXW + c * CHUNK

    def issue_idx(c, b):
        pltpu.async_copy(tags_ref.at[pl.ds(ibase(c), CHUNK)], idx_v.at[b], si[b])

    def wait_idx(b):
        pltpu.make_async_copy(tags_ref.at[pl.ds(0, CHUNK)], idx_v.at[b], si[b]).wait()

    def issue_out(c, b):
        pltpu.async_copy(out_v.at[b], out_ref.at[pl.ds(ibase(c) * D, CHUNK * D)], so[b])

    def wait_out(b):
        pltpu.make_async_copy(out_v.at[b], out_ref.at[pl.ds(0, CHUNK * D)], so[b]).wait()

    issue_idx(0, 0)
    issue_idx(1, 1)

    @pl.loop(0, NCHUNK // 2)
    def _super(s):
        for b in range(2):
            c = s * 2 + b

            wait_idx(b)

            @pl.when(c >= 2)
            def _():
                wait_out(b)

            @pl.loop(0, GROUPS, unroll=4)
            def _grp(g):
                iv = idx_v[b, pl.ds(g * 16, 16)]
                dst = g * (16 * D) + row_off
                for d in range(D):
                    vals = plsc.load_gather(
                        tab_v, [iv, jnp.full((16,), d, jnp.int32)]
                    )
                    plsc.store_scatter(out_v.at[b], [dst + d], vals)

            issue_out(c, b)

            @pl.when(c + 2 < NCHUNK)
            def _():
                issue_idx(c + 2, b)

    wait_out(0)
    wait_out(1)


@jax.jit
def _run(tags1d, embedding):
    mesh = plsc.VectorSubcoreMesh(
        core_axis_name="c", subcore_axis_name="s", num_cores=NC, num_subcores=NS
    )
    kern = pl.kernel(
        _body,
        out_type=jax.ShapeDtypeStruct((N * D,), jnp.float32),
        mesh=mesh,
        scratch_types=[
            pltpu.VMEM((VOCAB, D), jnp.float32),
            pltpu.VMEM((2, CHUNK), jnp.int32),
            pltpu.VMEM((2, CHUNK * D), jnp.float32),
            pltpu.SemaphoreType.DMA,
            pltpu.SemaphoreType.DMA,
            pltpu.SemaphoreType.DMA,
            pltpu.SemaphoreType.DMA,
        ],
        compiler_params=pltpu.CompilerParams(
            use_tc_tiling_on_sc=False, needs_layout_passes=False
        ),
    )
    return kern(tags1d, embedding)


def kernel(preprocessed_tags, embedding):
    tags1d = preprocessed_tags.reshape(N)
    out = _run(tags1d, embedding)
    return out.reshape(B, L, D)
